# baseline (speedup 1.0000x reference)
import functools

import jax
import jax.numpy as jnp
from jax import lax
from jax.experimental import pallas as pl
from jax.experimental.pallas import tpu as pltpu
from jax.experimental.pallas import tpu_sc as plsc

_NW = 32
_CHUNK_ROWS = 32


def kernel(x, x_len):
    B, L, D = x.shape
    CW = _CHUNK_ROWS * D
    WPW = B * L * D // _NW
    NCHUNK = WPW // CW
    assert WPW % CW == 0 and (B * L * D) % _NW == 0 and _NW % B == 0
    HALVES = _NW // B
    xlen = x_len.astype(jnp.int32)
    xf = x.reshape(-1)

    mesh = plsc.VectorSubcoreMesh(core_axis_name="c", subcore_axis_name="s")

    @functools.partial(
        pl.kernel,
        out_type=jax.ShapeDtypeStruct((B * L * D,), jnp.float32),
        mesh=mesh,
        scratch_types=[
            pltpu.VMEM((32,), jnp.int32),
            pltpu.VMEM((CW,), jnp.float32),
            pltpu.VMEM((CW,), jnp.float32),
        ],
    )
    def sqz(x_hbm, len_hbm, out_hbm, len_v, buf, zbuf):
        cid = lax.axis_index("c")
        sid = lax.axis_index("s")
        wid = sid * 2 + cid
        b = wid // HALVES
        h = wid % HALVES
        base = wid * WPW

        pltpu.sync_copy(len_hbm, len_v.at[pl.ds(0, 16)])
        l = len_v[pl.ds(b, 16)][0]
        vw = jnp.clip(l * D - h * WPW, 0, WPW)

        zeros16 = jnp.zeros((16,), jnp.float32)

        def zinit(t, carry):
            zbuf[pl.ds(t * 16, 16)] = zeros16
            return carry

        lax.fori_loop(0, CW // 16, zinit, 0)

        def body(ci, carry):
            off = ci * CW
            nv = jnp.clip(vw - off, 0, CW)

            @pl.when(nv > 0)
            def _():
                pltpu.sync_copy(x_hbm.at[pl.ds(base + off, CW)], buf)

                @pl.when(nv < CW)
                def _():
                    def ztail(t, c2):
                        buf[pl.ds(t * 16, 16)] = zeros16
                        return c2

                    lax.fori_loop(nv // 16, CW // 16, ztail, 0)

                pltpu.sync_copy(buf, out_hbm.at[pl.ds(base + off, CW)])

            @pl.when(nv == 0)
            def _():
                pltpu.sync_copy(zbuf, out_hbm.at[pl.ds(base + off, CW)])

            return carry

        lax.fori_loop(0, NCHUNK, body, 0)

    return sqz(xf, xlen).reshape(B, L, D)


# speedup vs baseline: 1.1511x; 1.1511x over previous
import functools

import jax
import jax.numpy as jnp
from jax import lax
from jax.experimental import pallas as pl
from jax.experimental.pallas import tpu as pltpu
from jax.experimental.pallas import tpu_sc as plsc

_NW = 32
_CHUNK_ROWS = 16
_NBUF = 4
_RAHEAD = 2


def kernel(x, x_len):
    B, L, D = x.shape
    CW = _CHUNK_ROWS * D
    NCHUNKS = B * L * D // CW
    CPB = L * D // CW
    NPW = NCHUNKS // _NW
    assert NCHUNKS % _NW == 0 and NPW % _NBUF == 0 and L % _CHUNK_ROWS == 0
    xlen = x_len.astype(jnp.int32)
    xf = x.reshape(-1)

    mesh = plsc.VectorSubcoreMesh(core_axis_name="c", subcore_axis_name="s")

    @functools.partial(
        pl.kernel,
        out_type=jax.ShapeDtypeStruct((B * L * D,), jnp.float32),
        mesh=mesh,
        scratch_types=(
            [pltpu.VMEM((32,), jnp.int32)]
            + [pltpu.VMEM((CW,), jnp.float32) for _ in range(_NBUF + 1)]
            + [pltpu.SemaphoreType.DMA for _ in range(2 * _NBUF)]
        ),
    )
    def sqz(x_hbm, len_hbm, out_hbm, len_v, *rest):
        bufs = rest[:_NBUF]
        zbuf = rest[_NBUF]
        rsem = rest[_NBUF + 1:_NBUF + 1 + _NBUF]
        wsem = rest[_NBUF + 1 + _NBUF:]

        cid = lax.axis_index("c")
        sid = lax.axis_index("s")
        wid = sid * 2 + cid

        pltpu.sync_copy(len_hbm, len_v.at[pl.ds(0, 16)])
        zeros16 = jnp.zeros((16,), jnp.float32)

        def valid_words(c):
            b = c // CPB
            l = len_v[pl.ds(b, 16)][0]
            woff = (c % CPB) * CW
            return jnp.clip(l * D - woff, 0, CW)

        def read_start(c, k):
            pltpu.make_async_copy(
                x_hbm.at[pl.ds(c * CW, CW)], bufs[k], rsem[k]).start()

        def read_wait(c, k):
            pltpu.make_async_copy(
                x_hbm.at[pl.ds(c * CW, CW)], bufs[k], rsem[k]).wait()

        def write_start(src, c, k):
            pltpu.make_async_copy(
                src, out_hbm.at[pl.ds(c * CW, CW)], wsem[k]).start()

        def write_drain(k):
            pltpu.make_async_copy(
                zbuf, out_hbm.at[pl.ds(wid * CW, CW)], wsem[k]).wait()

        def zinit(t, carry):
            zbuf[pl.ds(t * 16, 16)] = zeros16
            return carry

        lax.fori_loop(0, CW // 16, zinit, 0)

        def chunk_of(i):
            return i * _NW + wid

        for i in range(_RAHEAD):
            c = chunk_of(i)
            nv = valid_words(c)

            @pl.when(nv > 0)
            def _(c=c, i=i):
                read_start(c, i % _NBUF)

        def group(g, carry):
            for k in range(_NBUF):
                i = g * _NBUF + k
                i2 = i + _RAHEAD
                k2 = (k + _RAHEAD) % _NBUF

                @pl.when(i2 < NPW)
                def _():
                    c2 = chunk_of(i2)
                    nv2 = valid_words(c2)

                    @pl.when(i2 >= _NBUF)
                    def _():
                        write_drain(k2)

                    @pl.when(nv2 > 0)
                    def _():
                        read_start(c2, k2)

                c = chunk_of(i)
                nv = valid_words(c)

                @pl.when(nv > 0)
                def _():
                    read_wait(c, k)

                    @pl.when(nv < CW)
                    def _():
                        def ztail(t, c3):
                            bufs[k][pl.ds(t * 16, 16)] = zeros16
                            return c3

                        lax.fori_loop(nv // 16, CW // 16, ztail, 0)

                    write_start(bufs[k], c, k)

                @pl.when(nv == 0)
                def _():
                    write_start(zbuf, c, k)

            return carry

        lax.fori_loop(0, NPW // _NBUF, group, 0)

        for k in range(_NBUF):
            write_drain(k)

    return sqz(xf, xlen).reshape(B, L, D)


# speedup vs baseline: 4.3673x; 3.7940x over previous
import jax
import jax.numpy as jnp
from jax.experimental import pallas as pl
from jax.experimental.pallas import tpu as pltpu

_BL = 512


def kernel(x, x_len):
    B, L, D = x.shape
    nj = L // _BL
    nsteps = B * nj
    xlen = x_len.astype(jnp.int32)

    def body(xlen_ref, x_hbm, o_ref, buf, sem):
        b = pl.program_id(0)
        j = pl.program_id(1)
        k = b * nj + j
        slot = jax.lax.rem(k, 2)

        def needed(bb, jj):
            return jj * _BL < xlen_ref[bb]

        def issue(bb, jj, sl):
            pltpu.make_async_copy(
                x_hbm.at[bb, pl.ds(jj * _BL, _BL), :],
                buf.at[sl],
                sem.at[sl],
            ).start()

        @pl.when((k == 0) & needed(0, 0))
        def _():
            issue(0, 0, 0)

        k1 = k + 1
        b1 = k1 // nj
        j1 = jax.lax.rem(k1, nj)

        @pl.when((k1 < nsteps) & needed(b1, j1))
        def _():
            issue(b1, j1, jax.lax.rem(k1, 2))

        l = xlen_ref[b]
        start = j * _BL
        nv = jnp.clip(l - start, 0, _BL)

        @pl.when(nv == 0)
        def _():
            o_ref[...] = jnp.zeros_like(o_ref)

        @pl.when(nv > 0)
        def _():
            pltpu.make_async_copy(
                x_hbm.at[b, pl.ds(start, _BL), :], buf.at[slot], sem.at[slot]
            ).wait()

            @pl.when(nv == _BL)
            def _():
                o_ref[...] = buf[slot][None]

            @pl.when(nv < _BL)
            def _():
                row = jax.lax.broadcasted_iota(jnp.int32, (1, _BL, D), 1)
                o_ref[...] = jnp.where(row < nv, buf[slot][None], 0.0)

    grid_spec = pltpu.PrefetchScalarGridSpec(
        num_scalar_prefetch=1,
        grid=(B, nj),
        in_specs=[pl.BlockSpec(memory_space=pl.ANY)],
        out_specs=pl.BlockSpec((1, _BL, D), lambda b, j, xlen_ref: (b, j, 0)),
        scratch_shapes=[
            pltpu.VMEM((2, _BL, D), jnp.float32),
            pltpu.SemaphoreType.DMA((2,)),
        ],
    )
    return pl.pallas_call(
        body,
        grid_spec=grid_spec,
        out_shape=jax.ShapeDtypeStruct((B, L, D), x.dtype),
    )(xlen, x)
